# baseline (speedup 1.0000x reference)
import jax
import jax.numpy as jnp
from jax.experimental import pallas as pl
from jax.experimental.pallas import tpu as pltpu
from jax.experimental.pallas import tpu_sc as plsc

LATENT = 256
HID = 512
NL = 8
NPD = 2
NEXP = NPD ** 3
IN_DIM = 3
OUT_DIM = 1

TM = 512


def _vector_mesh():
    return plsc.VectorSubcoreMesh(core_axis_name="c", subcore_axis_name="s")


def _sc_reorder(data, pos_cur, pos_prev=None):
    T, D = data.shape
    WIN = 64
    NSUB = 32
    PER = T // NSUB
    NW = PER // WIN
    indexed = pos_prev is not None

    def body(*args):
        if indexed:
            data_hbm, pc_hbm, pp_hbm, o_hbm, buf, pidx, gsem, ssem = args
        else:
            data_hbm, pc_hbm, o_hbm, buf, pidx, gsem, ssem = args
        c = jax.lax.axis_index("c")
        s = jax.lax.axis_index("s")
        base = (c * 16 + s) * PER
        pltpu.sync_copy(pc_hbm.at[0, pl.ds(base, PER)], pidx.at[0])
        if indexed:
            pltpu.sync_copy(pp_hbm.at[0, pl.ds(base, PER)], pidx.at[1])

        def gather_copy(w):
            if indexed:
                src = data_hbm.at[pidx.at[1, pl.ds(w * WIN, WIN)]]
            else:
                src = data_hbm.at[pl.ds(base + w * WIN, WIN), :]
            return pltpu.make_async_copy(src, buf.at[w % 2], gsem.at[w % 2])

        def scatter_copy(w):
            dst = o_hbm.at[pidx.at[0, pl.ds(w * WIN, WIN)]]
            return pltpu.make_async_copy(buf.at[w % 2], dst, ssem.at[w % 2])

        g = [gather_copy(w) for w in range(NW)]
        sc = [scatter_copy(w) for w in range(NW)]
        g[0].start()
        if NW > 1:
            g[1].start()
        for w in range(NW):
            g[w].wait()
            sc[w].start()
            if w + 2 < NW:
                sc[w].wait()
                g[w + 2].start()
        for w in range(max(0, NW - 2), NW):
            sc[w].wait()

    scratch = [pltpu.VMEM((2, WIN, D), data.dtype),
               pltpu.VMEM((2 if indexed else 1, PER), jnp.int32),
               pltpu.SemaphoreType.DMA((2,)),
               pltpu.SemaphoreType.DMA((2,))]
    k = pl.kernel(body,
                  out_type=jax.ShapeDtypeStruct((T, D), data.dtype),
                  mesh=_vector_mesh(),
                  scratch_types=scratch)
    if indexed:
        return k(data, pos_cur, pos_prev)
    return k(data, pos_cur)


def _sc_permute(data, pos_prev, pos_cur):
    return _sc_reorder(data, pos_cur, pos_prev=pos_prev)


def _sc_scatter(data, pos_cur):
    return _sc_reorder(data, pos_cur)


def _sc_gather(data, pos):
    T, D = data.shape
    WIN = 128

    @pl.kernel(out_type=jax.ShapeDtypeStruct((T, D), data.dtype),
               mesh=_vector_mesh())
    def k(data_hbm, p_hbm, o_hbm):
        def body(p_vmem, o_vmem):
            pltpu.sync_copy(data_hbm.at[p_vmem.at[0]], o_vmem)

        pltpu.emit_pipeline(
            body,
            grid=(T // WIN,),
            in_specs=[pl.BlockSpec((1, WIN), lambda i: (0, i))],
            out_specs=[pl.BlockSpec((WIN, D), lambda i: (i, 0))],
            core_axis_name=("c", "s"),
            dimension_semantics=(pltpu.PARALLEL,),
        )(p_hbm, o_hbm)

    return k(data, pos)


def _tc_pair(A_list, WA_list, bA, WB, bB, meta, reluB):
    T = A_list[0].shape[0]
    NA = WA_list[0].shape[2]
    NB = WB.shape[2]
    nA = len(A_list)
    nb = T // TM

    def body(*refs):
        eloA_r, ehiA_r, eloB_r, ehiB_r = refs[:4]
        tidA_r, tidB_r = refs[4:6]
        a_refs = refs[6:6 + nA]
        wa_refs = refs[6 + nA:6 + 2 * nA]
        bA_r, wB_r, bB_r, o_ref, accA_r = refs[6 + 2 * nA:]
        m = pl.program_id(0)
        tidA = tidA_r[...]
        tidB = tidB_r[...]

        accA_r[...] = jnp.zeros_like(accA_r)

        def stepA(e, c):
            part = jnp.dot(a_refs[0][...], wa_refs[0][e],
                           preferred_element_type=jnp.float32)
            for a_r, w_r in zip(a_refs[1:], wa_refs[1:]):
                part = part + jnp.dot(a_r[...], w_r[e],
                                      preferred_element_type=jnp.float32)
            accA_r[...] = jnp.where(tidA == e, part, accA_r[...])
            return c

        jax.lax.fori_loop(eloA_r[m], ehiA_r[m] + 1, stepA, 0)
        accA_r[...] = jnp.maximum(accA_r[...] + bA_r[...], 0.0)

        def stepB(e, c):
            part = jnp.dot(accA_r[...], wB_r[e],
                           preferred_element_type=jnp.float32)
            o_ref[...] = jnp.where(tidB == e, part, o_ref[...])
            return c

        jax.lax.fori_loop(eloB_r[m], ehiB_r[m] + 1, stepB, 0)
        out = o_ref[...] + bB_r[...]
        if reluB:
            out = jnp.maximum(out, 0.0)
        o_ref[...] = out

    in_specs = [pl.BlockSpec((TM, 1), lambda m, *s: (m, 0)),
                pl.BlockSpec((TM, 1), lambda m, *s: (m, 0))]
    for A in A_list:
        K = A.shape[1]
        in_specs.append(pl.BlockSpec((TM, K), lambda m, *s: (m, 0)))
    for W in WA_list:
        in_specs.append(pl.BlockSpec(W.shape, lambda m, *s: (0, 0, 0)))
    in_specs.append(pl.BlockSpec((1, NA), lambda m, *s: (0, 0)))
    in_specs.append(pl.BlockSpec(WB.shape, lambda m, *s: (0, 0, 0)))
    in_specs.append(pl.BlockSpec((1, NB), lambda m, *s: (0, 0)))

    grid_spec = pltpu.PrefetchScalarGridSpec(
        num_scalar_prefetch=4,
        grid=(nb,),
        in_specs=in_specs,
        out_specs=pl.BlockSpec((TM, NB), lambda m, *s: (m, 0)),
        scratch_shapes=[pltpu.VMEM((TM, NA), jnp.float32)],
    )
    return pl.pallas_call(
        body,
        grid_spec=grid_spec,
        out_shape=jax.ShapeDtypeStruct((T, NB), jnp.float32),
    )(meta["eloA"], meta["ehiA"], meta["eloB"], meta["ehiB"],
      meta["tidA"], meta["tidB"], *A_list, *WA_list, bA, WB, bB)


def _routing_pairs(xyz_f):
    T = xyz_f.shape[0]
    NBK = NEXP * NEXP
    BLK = 128
    NBLK = T // BLK
    nb = T // TM
    tril = jnp.tril(jnp.ones((BLK, BLK), jnp.float32), k=-1)
    bids = jnp.arange(NBK, dtype=jnp.int32)
    r_iota = jnp.arange(T, dtype=jnp.float32)
    metas = []
    for p in range(NL // 2):
        tids = []
        for i in (2 * p, 2 * p + 1):
            alpha = 2.0 ** (i + 1)
            t3 = jnp.floor(alpha * xyz_f).astype(jnp.int32) % NPD
            tids.append(t3[:, 0] + NPD * t3[:, 1] + NPD ** 2 * t3[:, 2])
        key = tids[0] * NEXP + tids[1]
        oh = (key[:, None] == bids[None, :]).astype(jnp.float32)
        oh3 = oh.reshape(NBLK, BLK, NBK)
        intra = jnp.einsum("lk,bkc->blc", tril, oh3)
        bs = jnp.sum(oh3, axis=1)
        blockoff = jnp.cumsum(bs, axis=0) - bs
        counts = jnp.sum(bs, axis=0)
        cum = jnp.cumsum(counts)
        offs = cum - counts
        pos3 = intra + blockoff[:, None, :] + offs[None, None, :]
        pos = jnp.sum(pos3 * oh3, axis=2).reshape(T).astype(jnp.int32)
        bucket_s = jnp.sum(
            (r_iota[:, None] >= cum[None, :]).astype(jnp.int32), axis=1)
        tidA_s = bucket_s // NEXP
        tidB_s = bucket_s % NEXP
        tB = tidB_s.reshape(nb, TM)
        metas.append(dict(
            pos=pos.reshape(1, T),
            tidA=tidA_s.reshape(T, 1),
            tidB=tidB_s.reshape(T, 1),
            eloA=tidA_s[0::TM],
            ehiA=tidA_s[TM - 1::TM],
            eloB=jnp.min(tB, axis=1),
            ehiB=jnp.max(tB, axis=1)))
    return metas


def _pad_cols(a, to):
    pad = to - a.shape[-1]
    if pad == 0:
        return a
    cfg = [(0, 0)] * (a.ndim - 1) + [(0, pad)]
    return jnp.pad(a, cfg)


def _pad_rows(w, to):
    pad = to - w.shape[1]
    if pad == 0:
        return w
    return jnp.pad(w, [(0, 0), (0, pad), (0, 0)])


def kernel(lat, xyz, W0, W1, W2, W3, W4, W5, W6, W7,
           b0, b1, b2, b3, b4, b5, b6, b7):
    B, N, _ = xyz.shape
    T = B * N
    batch_shape = xyz.shape[:-1]
    XF = LATENT + IN_DIM
    XP = 384
    SKIP = HID - XF
    SKIPP = 256

    xyz_f = xyz.reshape(T, IN_DIM)
    x = jnp.concatenate(
        [jnp.broadcast_to(lat, batch_shape + (LATENT,)), xyz],
        axis=-1).reshape(T, XF)
    x = _pad_cols(x, XP)

    m0, m1, m2, m3 = _routing_pairs(xyz_f)

    W0p = _pad_rows(W0, XP)
    W3p = _pad_cols(W3, SKIPP)
    b3p = _pad_cols(b3, SKIPP)
    W4a = _pad_rows(W4[:, :SKIP, :], SKIPP)
    W4b = _pad_rows(W4[:, SKIP:, :], XP)
    W7p = _pad_cols(W7, 128)
    b7p = _pad_cols(b7, 128)

    x_s0 = _sc_scatter(x, m0["pos"])
    h1 = _tc_pair([x_s0], [W0p], b0, W1, b1, m0, reluB=True)
    h1p = _sc_permute(h1, m0["pos"], m1["pos"])
    h3 = _tc_pair([h1p], [W2], b2, W3p, b3p, m1, reluB=True)
    h3p = _sc_permute(h3, m1["pos"], m2["pos"])
    x_s4 = _sc_scatter(x, m2["pos"])
    h5 = _tc_pair([h3p, x_s4], [W4a, W4b], b4, W5, b5, m2, reluB=True)
    h5p = _sc_permute(h5, m2["pos"], m3["pos"])
    out7 = _tc_pair([h5p], [W6], b6, W7p, b7p, m3, reluB=False)
    y = _sc_gather(out7, m3["pos"])
    return y[:, :OUT_DIM].reshape(batch_shape + (OUT_DIM,))
